# baseline (speedup 1.0000x reference)
import functools

import jax
import jax.numpy as jnp
from jax import lax
from jax.experimental import pallas as pl
from jax.experimental.pallas import tpu as pltpu
from jax.experimental.pallas import tpu_sc as plsc

N = 10000
E = 320000
D = 128
NC = 2
NS = 16
NT = NC * NS
CHUNK = 128
CHUNKS = 80
EPT = CHUNKS * CHUNK
EP = NT * EPT
NPAD = 10112
RPT = NPAD // NS

_mesh = plsc.VectorSubcoreMesh(core_axis_name="c", subcore_axis_name="s")


WOUT = 8


def _deg_body(ones_hbm, rows_hbm, zero_hbm, degp_hbm, idx_v, ones_v, d_sh, sem):
    cid = lax.axis_index("c")
    sid = lax.axis_index("s")
    wid = sid * NC + cid
    pltpu.sync_copy(zero_hbm.at[pl.ds(sid * RPT, RPT)], d_sh.at[pl.ds(sid * RPT, RPT)])
    pltpu.sync_copy(ones_hbm, ones_v)
    pltpu.sync_copy(rows_hbm.at[wid], idx_v)
    plsc.subcore_barrier()

    @pl.loop(0, CHUNKS)
    def _(j):
        pltpu.async_copy(ones_v, d_sh.at[idx_v.at[j]], sem, add=True)

        @pl.when(j >= WOUT)
        def _():
            pltpu.make_async_copy(ones_v, d_sh.at[idx_v.at[j - WOUT]], sem).wait()

    @pl.loop(0, WOUT)
    def _(j):
        pltpu.make_async_copy(ones_v, d_sh.at[idx_v.at[CHUNKS - WOUT + j]], sem).wait()

    plsc.subcore_barrier()
    pltpu.sync_copy(d_sh.at[pl.ds(sid * RPT, RPT)],
                    degp_hbm.at[cid, pl.ds(sid * RPT, RPT)])


_deg_call = pl.kernel(
    _deg_body,
    out_type=jax.ShapeDtypeStruct((NC, NPAD, D), jnp.float32),
    mesh=_mesh,
    scratch_types=[
        pltpu.VMEM((CHUNKS, CHUNK), jnp.int32),
        pltpu.VMEM((CHUNK, D), jnp.float32),
        pltpu.VMEM_SHARED((NPAD, D), jnp.float32),
        pltpu.SemaphoreType.DMA,
    ],
)


def _layer_body(g_hbm, rows_hbm, cols_hbm, zero_hbm, sp_hbm,
                row_v, cw, buf0, buf1, s_sh,
                semg0, semg1, semc0, semc1, sems0, sems1):
    cid = lax.axis_index("c")
    sid = lax.axis_index("s")
    wid = sid * NC + cid
    pltpu.sync_copy(zero_hbm.at[pl.ds(sid * RPT, RPT)], s_sh.at[pl.ds(sid * RPT, RPT)])
    pltpu.sync_copy(rows_hbm.at[wid], row_v)
    pltpu.sync_copy(cols_hbm.at[wid, 0], cw.at[0])
    pltpu.sync_copy(cols_hbm.at[wid, 1], cw.at[1])
    plsc.subcore_barrier()

    pltpu.async_copy(g_hbm.at[cw.at[0]], buf0, semg0)
    pltpu.async_copy(g_hbm.at[cw.at[1]], buf1, semg1)

    @pl.loop(0, CHUNKS, step=2)
    def _(jj):
        pltpu.make_async_copy(g_hbm.at[cw.at[0]], buf0, semg0).wait()

        @pl.when(jj + 2 < CHUNKS)
        def _():
            pltpu.async_copy(cols_hbm.at[wid, jj + 2], cw.at[0], semc0)

        pltpu.async_copy(buf0, s_sh.at[row_v.at[jj]], sems0, add=True)

        pltpu.make_async_copy(g_hbm.at[cw.at[1]], buf1, semg1).wait()

        @pl.when(jj + 3 < CHUNKS)
        def _():
            pltpu.async_copy(cols_hbm.at[wid, jj + 3], cw.at[1], semc1)

        pltpu.async_copy(buf1, s_sh.at[row_v.at[jj + 1]], sems1, add=True)

        @pl.when(jj + 2 < CHUNKS)
        def _():
            pltpu.make_async_copy(buf0, s_sh.at[row_v.at[jj]], sems0).wait()
            pltpu.make_async_copy(cols_hbm.at[wid, jj + 2], cw.at[0], semc0).wait()
            pltpu.async_copy(g_hbm.at[cw.at[0]], buf0, semg0)

        @pl.when(jj + 3 < CHUNKS)
        def _():
            pltpu.make_async_copy(buf1, s_sh.at[row_v.at[jj + 1]], sems1).wait()
            pltpu.make_async_copy(cols_hbm.at[wid, jj + 3], cw.at[1], semc1).wait()
            pltpu.async_copy(g_hbm.at[cw.at[1]], buf1, semg1)

    pltpu.make_async_copy(buf0, s_sh.at[row_v.at[CHUNKS - 2]], sems0).wait()
    pltpu.make_async_copy(buf1, s_sh.at[row_v.at[CHUNKS - 1]], sems1).wait()
    plsc.subcore_barrier()
    pltpu.sync_copy(s_sh.at[pl.ds(sid * RPT, RPT)],
                    sp_hbm.at[cid, pl.ds(sid * RPT, RPT)])


_layer_call = pl.kernel(
    _layer_body,
    out_type=jax.ShapeDtypeStruct((NC, NPAD, D), jnp.float32),
    mesh=_mesh,
    scratch_types=[
        pltpu.VMEM((CHUNKS, CHUNK), jnp.int32),
        pltpu.VMEM((2, CHUNK), jnp.int32),
        pltpu.VMEM((CHUNK, D), jnp.float32),
        pltpu.VMEM((CHUNK, D), jnp.float32),
        pltpu.VMEM_SHARED((NPAD, D), jnp.float32),
        pltpu.SemaphoreType.DMA,
        pltpu.SemaphoreType.DMA,
        pltpu.SemaphoreType.DMA,
        pltpu.SemaphoreType.DMA,
        pltpu.SemaphoreType.DMA,
        pltpu.SemaphoreType.DMA,
    ],
)


def _prep_body(degp_ref, x_ref, g0_ref, invdeg_ref, sd_ref):
    deg = degp_ref[0, :N, 0:1] + degp_ref[1, :N, 0:1]
    a = jnp.where(deg > 0, lax.rsqrt(deg), 0.0)
    invdeg_ref[...] = a * a
    sd_ref[...] = deg * a
    g0_ref[...] = x_ref[...] * a


def _combine_body(sp_ref, invdeg_ref, gprev_ref, g_ref, gsum_ref):
    g = (sp_ref[0, :N, :] + sp_ref[1, :N, :]) * invdeg_ref[...]
    g_ref[...] = g
    gsum_ref[...] = gprev_ref[...] + g


def _final_body(sp_ref, invdeg_ref, gsum_ref, x_ref, sd_ref, w_ref, b_ref, out_ref):
    g3 = (sp_ref[0, :N, :] + sp_ref[1, :N, :]) * invdeg_ref[...]
    gs = gsum_ref[...] + g3
    f = (x_ref[...] + sd_ref[...] * gs) * 0.25
    out_ref[...] = lax.dot_general(
        f, w_ref[...], (((1,), (1,)), ((), ())),
        preferred_element_type=jnp.float32) + b_ref[...]


_f32 = jnp.float32


def kernel(x, edge_index, W_out, b_out):
    row = edge_index[0].astype(jnp.int32)
    col = edge_index[1].astype(jnp.int32)
    npad_e = EP - E
    pad_ar = jnp.arange(npad_e, dtype=jnp.int32)
    rows3 = jnp.concatenate([row, N + (pad_ar % 16)]).reshape(NT, CHUNKS, CHUNK)
    cols3 = jnp.concatenate([col, pad_ar % N]).reshape(NT, CHUNKS, CHUNK)
    zeroD = jnp.zeros((NPAD, D), _f32)
    onesD = jnp.ones((CHUNK, D), _f32)

    degp = _deg_call(onesD, rows3, zeroD)

    g0, invdeg, sd = pl.pallas_call(
        _prep_body,
        out_shape=[
            jax.ShapeDtypeStruct((N, D), _f32),
            jax.ShapeDtypeStruct((N, 1), _f32),
            jax.ShapeDtypeStruct((N, 1), _f32),
        ],
    )(degp, x)

    combine = pl.pallas_call(
        _combine_body,
        out_shape=[
            jax.ShapeDtypeStruct((N, D), _f32),
            jax.ShapeDtypeStruct((N, D), _f32),
        ],
    )

    sp1 = _layer_call(g0, rows3, cols3, zeroD)
    g1, gs1 = combine(sp1, invdeg, jnp.zeros((N, D), _f32))
    sp2 = _layer_call(g1, rows3, cols3, zeroD)
    g2, gs2 = combine(sp2, invdeg, gs1)
    sp3 = _layer_call(g2, rows3, cols3, zeroD)

    out = pl.pallas_call(
        _final_body,
        out_shape=jax.ShapeDtypeStruct((N, D), _f32),
    )(sp3, invdeg, gs2, x, sd, W_out, b_out.reshape(1, D))
    return out


# speedup vs baseline: 1.3839x; 1.3839x over previous
import functools

import jax
import jax.numpy as jnp
from jax import lax
from jax.experimental import pallas as pl
from jax.experimental.pallas import tpu as pltpu
from jax.experimental.pallas import tpu_sc as plsc

N = 10000
E = 320000
D = 128
NC = 2
NS = 16
NT = NC * NS
CHUNK = 128
CHUNKS = 80
EPT = CHUNKS * CHUNK
EP = NT * EPT
NPAD = 10112
RPT = NPAD // NS

_mesh = plsc.VectorSubcoreMesh(core_axis_name="c", subcore_axis_name="s")


NP2 = 10240


def _deg_body(rows_hbm, degp_hbm, idx_v, ones1, z1, d1, sem):
    cid = lax.axis_index("c")
    sid = lax.axis_index("s")
    wid = sid * NC + cid
    @pl.loop(0, CHUNK // 16)
    def _(i):
        ones1[pl.ds(i * 16, 16)] = jnp.ones((16,), jnp.float32)

    @pl.loop(0, NP2 // NS // 16)
    def _(i):
        z1[pl.ds(i * 16, 16)] = jnp.zeros((16,), jnp.float32)

    pltpu.sync_copy(rows_hbm.at[wid], idx_v)
    pltpu.sync_copy(z1, d1.at[pl.ds(sid * (NP2 // NS), NP2 // NS)])
    plsc.subcore_barrier()

    @pl.loop(0, CHUNKS)
    def _(j):
        pltpu.sync_copy(ones1, d1.at[idx_v.at[j]], add=True)

    plsc.subcore_barrier()

    @pl.when(sid == 0)
    def _():
        pltpu.sync_copy(d1, degp_hbm.at[cid])


_deg_call = pl.kernel(
    _deg_body,
    out_type=jax.ShapeDtypeStruct((NC, NP2), jnp.float32),
    mesh=_mesh,
    scratch_types=[
        pltpu.VMEM((CHUNKS, CHUNK), jnp.int32),
        pltpu.VMEM((CHUNK,), jnp.float32),
        pltpu.VMEM((NP2 // NS,), jnp.float32),
        pltpu.VMEM_SHARED((NP2,), jnp.float32),
        pltpu.SemaphoreType.DMA,
    ],
)


def _layer_body(g_hbm, rows_hbm, cols_hbm, zero_hbm, sp_hbm,
                row_v, cw, buf0, buf1, s_sh, semg0, semg1, semc0, semc1):
    cid = lax.axis_index("c")
    sid = lax.axis_index("s")
    wid = sid * NC + cid
    pltpu.sync_copy(zero_hbm.at[pl.ds(sid * RPT, RPT)], s_sh.at[pl.ds(sid * RPT, RPT)])
    pltpu.sync_copy(rows_hbm.at[wid], row_v)
    pltpu.sync_copy(cols_hbm.at[wid, 0], cw.at[0])
    pltpu.sync_copy(cols_hbm.at[wid, 1], cw.at[1])
    plsc.subcore_barrier()

    pltpu.async_copy(g_hbm.at[cw.at[0]], buf0, semg0)
    pltpu.async_copy(g_hbm.at[cw.at[1]], buf1, semg1)

    @pl.loop(0, CHUNKS, step=2)
    def _(jj):
        pltpu.make_async_copy(g_hbm.at[cw.at[0]], buf0, semg0).wait()

        @pl.when(jj + 2 < CHUNKS)
        def _():
            pltpu.async_copy(cols_hbm.at[wid, jj + 2], cw.at[0], semc0)

        pltpu.sync_copy(buf0, s_sh.at[row_v.at[jj]], add=True)

        @pl.when(jj + 2 < CHUNKS)
        def _():
            pltpu.make_async_copy(cols_hbm.at[wid, jj + 2], cw.at[0], semc0).wait()
            pltpu.async_copy(g_hbm.at[cw.at[0]], buf0, semg0)

        pltpu.make_async_copy(g_hbm.at[cw.at[1]], buf1, semg1).wait()

        @pl.when(jj + 3 < CHUNKS)
        def _():
            pltpu.async_copy(cols_hbm.at[wid, jj + 3], cw.at[1], semc1)

        pltpu.sync_copy(buf1, s_sh.at[row_v.at[jj + 1]], add=True)

        @pl.when(jj + 3 < CHUNKS)
        def _():
            pltpu.make_async_copy(cols_hbm.at[wid, jj + 3], cw.at[1], semc1).wait()
            pltpu.async_copy(g_hbm.at[cw.at[1]], buf1, semg1)

    plsc.subcore_barrier()
    pltpu.sync_copy(s_sh.at[pl.ds(sid * RPT, RPT)],
                    sp_hbm.at[cid, pl.ds(sid * RPT, RPT)])


_layer_call = pl.kernel(
    _layer_body,
    out_type=jax.ShapeDtypeStruct((NC, NPAD, D), jnp.float32),
    mesh=_mesh,
    scratch_types=[
        pltpu.VMEM((CHUNKS, CHUNK), jnp.int32),
        pltpu.VMEM((2, CHUNK), jnp.int32),
        pltpu.VMEM((CHUNK, D), jnp.float32),
        pltpu.VMEM((CHUNK, D), jnp.float32),
        pltpu.VMEM_SHARED((NPAD, D), jnp.float32),
        pltpu.SemaphoreType.DMA,
        pltpu.SemaphoreType.DMA,
        pltpu.SemaphoreType.DMA,
        pltpu.SemaphoreType.DMA,
    ],
)


def _prep_body(degp_ref, x_ref, g0_ref, invdeg_ref, sd_ref):
    deg = degp_ref[0, :N, 0:1] + degp_ref[1, :N, 0:1]
    a = jnp.where(deg > 0, lax.rsqrt(deg), 0.0)
    invdeg_ref[...] = a * a
    sd_ref[...] = deg * a
    g0_ref[...] = x_ref[...] * a


def _combine_body(sp_ref, invdeg_ref, gprev_ref, g_ref, gsum_ref):
    g = (sp_ref[0, :N, :] + sp_ref[1, :N, :]) * invdeg_ref[...]
    g_ref[...] = g
    gsum_ref[...] = gprev_ref[...] + g


def _final_body(sp_ref, invdeg_ref, gsum_ref, x_ref, sd_ref, w_ref, b_ref, out_ref):
    g3 = (sp_ref[0, :N, :] + sp_ref[1, :N, :]) * invdeg_ref[...]
    gs = gsum_ref[...] + g3
    f = (x_ref[...] + sd_ref[...] * gs) * 0.25
    out_ref[...] = lax.dot_general(
        f, w_ref[...], (((1,), (1,)), ((), ())),
        preferred_element_type=jnp.float32) + b_ref[...]


_f32 = jnp.float32


def kernel(x, edge_index, W_out, b_out):
    row = edge_index[0].astype(jnp.int32)
    col = edge_index[1].astype(jnp.int32)
    npad_e = EP - E
    pad_ar = jnp.arange(npad_e, dtype=jnp.int32)
    rows3 = jnp.concatenate([row, N + (pad_ar % 16)]).reshape(NT, CHUNKS, CHUNK)
    cols3 = jnp.concatenate([col, pad_ar % N]).reshape(NT, CHUNKS, CHUNK)
    zeroD = jnp.zeros((NPAD, D), _f32)

    degp = _deg_call(rows3)[:, :NPAD].reshape(NC, NPAD, 1)

    g0, invdeg, sd = pl.pallas_call(
        _prep_body,
        out_shape=[
            jax.ShapeDtypeStruct((N, D), _f32),
            jax.ShapeDtypeStruct((N, 1), _f32),
            jax.ShapeDtypeStruct((N, 1), _f32),
        ],
    )(degp, x)

    combine = pl.pallas_call(
        _combine_body,
        out_shape=[
            jax.ShapeDtypeStruct((N, D), _f32),
            jax.ShapeDtypeStruct((N, D), _f32),
        ],
    )

    sp1 = _layer_call(g0, rows3, cols3, zeroD)
    g1, gs1 = combine(sp1, invdeg, jnp.zeros((N, D), _f32))
    sp2 = _layer_call(g1, rows3, cols3, zeroD)
    g2, gs2 = combine(sp2, invdeg, gs1)
    sp3 = _layer_call(g2, rows3, cols3, zeroD)

    out = pl.pallas_call(
        _final_body,
        out_shape=jax.ShapeDtypeStruct((N, D), _f32),
    )(sp3, invdeg, gs2, x, sd, W_out, b_out.reshape(1, D))
    return out
